# baseline (speedup 1.0000x reference)
import jax
import jax.numpy as jnp
from jax.experimental import pallas as pl

HEADS = 16
BUCKETS = 64
DIM = 64
SEQ = 8192
PACK = 2 * DIM
PROWS = SEQ * DIM // PACK
PB = PROWS // BUCKETS


def _body(q_ref, k_ref, pq_ref, pk_ref, out_ref):
    inv = 1.0 / (SEQ // BUCKETS)

    def bucket_means(ref, pos):
        x = ref[0].reshape(BUCKETS, PB // 8, 8, PACK)
        s = x.sum(axis=1).sum(axis=1)
        return (s[:, :DIM] + s[:, DIM:]) * inv + pos

    sq = bucket_means(q_ref, pq_ref[0, 0])
    sk = bucket_means(k_ref, pk_ref[0, 0])
    r = jax.lax.dot_general(
        sq, sk, (((1,), (1,)), ((), ())), preferred_element_type=jnp.float32
    )
    r = r - jnp.max(r, axis=-1, keepdims=True)
    e = jnp.exp(r)
    out_ref[0] = e / jnp.sum(e, axis=-1, keepdims=True)


def kernel(q, k, q_pos_emb, k_pos_emb):
    bh = q.shape[0]
    qp = q.reshape(bh, PROWS, PACK)
    kp = k.reshape(bh, PROWS, PACK)
    return pl.pallas_call(
        _body,
        grid=(bh,),
        in_specs=[
            pl.BlockSpec((1, PROWS, PACK), lambda i: (i, 0, 0)),
            pl.BlockSpec((1, PROWS, PACK), lambda i: (i, 0, 0)),
            pl.BlockSpec((1, 1, BUCKETS, DIM), lambda i: (0, i % HEADS, 0, 0)),
            pl.BlockSpec((1, 1, BUCKETS, DIM), lambda i: (0, i % HEADS, 0, 0)),
        ],
        out_specs=pl.BlockSpec((1, BUCKETS, BUCKETS), lambda i: (i, 0, 0)),
        out_shape=jax.ShapeDtypeStruct((bh, BUCKETS, BUCKETS), jnp.float32),
    )(qp, kp, q_pos_emb, k_pos_emb)


# speedup vs baseline: 5.5433x; 5.5433x over previous
import jax
import jax.numpy as jnp
from jax.experimental import pallas as pl

HEADS = 16
BUCKETS = 64
DIM = 64
SEQ = 8192
RPB = SEQ // BUCKETS


def _body(q_ref, k_ref, w_ref, pq_ref, pk_ref, out_ref):
    w = w_ref[...]
    a = jax.lax.dot_general(
        q_ref[0], w, (((1,), (0,)), ((), ())), preferred_element_type=jnp.float32
    ) + pq_ref[0]
    b = jax.lax.dot_general(
        k_ref[0], w, (((1,), (0,)), ((), ())), preferred_element_type=jnp.float32
    ) + pk_ref[0]
    r = jax.lax.dot_general(
        a, b, (((0,), (0,)), ((), ())), preferred_element_type=jnp.float32
    )
    r = r - jnp.max(r, axis=-1, keepdims=True)
    e = jnp.exp(r)
    out_ref[0] = e / jnp.sum(e, axis=-1, keepdims=True)


def kernel(q, k, q_pos_emb, k_pos_emb):
    bh = q.shape[0]
    qt = jnp.swapaxes(q, 1, 2)
    kt = jnp.swapaxes(k, 1, 2)
    pqt = jnp.swapaxes(q_pos_emb[0], 1, 2)
    pkt = jnp.swapaxes(k_pos_emb[0], 1, 2)
    w = jnp.repeat(jnp.eye(BUCKETS, dtype=jnp.float32) / RPB, RPB, axis=0)
    return pl.pallas_call(
        _body,
        grid=(bh,),
        in_specs=[
            pl.BlockSpec((1, DIM, SEQ), lambda i: (i, 0, 0)),
            pl.BlockSpec((1, DIM, SEQ), lambda i: (i, 0, 0)),
            pl.BlockSpec((SEQ, BUCKETS), lambda i: (0, 0)),
            pl.BlockSpec((1, DIM, BUCKETS), lambda i: (i % HEADS, 0, 0)),
            pl.BlockSpec((1, DIM, BUCKETS), lambda i: (i % HEADS, 0, 0)),
        ],
        out_specs=pl.BlockSpec((1, BUCKETS, BUCKETS), lambda i: (i, 0, 0)),
        out_shape=jax.ShapeDtypeStruct((bh, BUCKETS, BUCKETS), jnp.float32),
    )(qt, kt, w, pqt, pkt)


# speedup vs baseline: 5.5715x; 1.0051x over previous
import jax
import jax.numpy as jnp
from jax.experimental import pallas as pl

HEADS = 16
BUCKETS = 64
DIM = 64
SEQ = 8192
RPB = SEQ // BUCKETS
HSEQ = SEQ // 2
HBUCK = BUCKETS // 2


def _body(q0_ref, q1_ref, k0_ref, k1_ref, w_ref, pq_ref, pk_ref, out_ref):
    w = w_ref[...]

    def means(lo_ref, hi_ref, pos):
        lo = jax.lax.dot_general(
            lo_ref[0], w, (((1,), (0,)), ((), ())),
            preferred_element_type=jnp.float32,
        )
        hi = jax.lax.dot_general(
            hi_ref[0], w, (((1,), (0,)), ((), ())),
            preferred_element_type=jnp.float32,
        )
        return jnp.concatenate((lo, hi), axis=1) + pos

    a = means(q0_ref, q1_ref, pq_ref[0])
    b = means(k0_ref, k1_ref, pk_ref[0])
    r = jax.lax.dot_general(
        a, b, (((0,), (0,)), ((), ())), preferred_element_type=jnp.float32
    )
    r = r - jnp.max(r, axis=-1, keepdims=True)
    e = jnp.exp(r)
    out_ref[0] = e / jnp.sum(e, axis=-1, keepdims=True)


def kernel(q, k, q_pos_emb, k_pos_emb):
    bh = q.shape[0]
    qt = jnp.swapaxes(q, 1, 2)
    kt = jnp.swapaxes(k, 1, 2)
    pqt = jnp.swapaxes(q_pos_emb[0], 1, 2)
    pkt = jnp.swapaxes(k_pos_emb[0], 1, 2)
    w = jnp.repeat(jnp.eye(HBUCK, dtype=jnp.float32) / RPB, RPB, axis=0)
    half = pl.BlockSpec((1, DIM, HSEQ), lambda i, lo=0: (i, 0, lo))
    return pl.pallas_call(
        _body,
        grid=(bh,),
        in_specs=[
            pl.BlockSpec((1, DIM, HSEQ), lambda i: (i, 0, 0)),
            pl.BlockSpec((1, DIM, HSEQ), lambda i: (i, 0, 1)),
            pl.BlockSpec((1, DIM, HSEQ), lambda i: (i, 0, 0)),
            pl.BlockSpec((1, DIM, HSEQ), lambda i: (i, 0, 1)),
            pl.BlockSpec((HSEQ, HBUCK), lambda i: (0, 0)),
            pl.BlockSpec((1, DIM, BUCKETS), lambda i: (i % HEADS, 0, 0)),
            pl.BlockSpec((1, DIM, BUCKETS), lambda i: (i % HEADS, 0, 0)),
        ],
        out_specs=pl.BlockSpec((1, BUCKETS, BUCKETS), lambda i: (i, 0, 0)),
        out_shape=jax.ShapeDtypeStruct((bh, BUCKETS, BUCKETS), jnp.float32),
    )(qt, qt, kt, kt, w, pqt, pkt)
